# baseline (speedup 1.0000x reference)
import functools

import jax
import jax.numpy as jnp
from jax import lax
from jax.experimental import pallas as pl
from jax.experimental.pallas import tpu as pltpu
from jax.experimental.pallas import tpu_sc as plsc

NC = 2
NS = 16
NW = NC * NS
BZ = 256
L = 16


def _w_convert(WT, Wtail):
    D, V = WT.shape
    n_full = V // 128
    rem = V - n_full * 128
    n_iter = (n_full + NW - 1) // NW

    mesh = plsc.VectorSubcoreMesh(
        core_axis_name="c", subcore_axis_name="s",
        num_cores=NC, num_subcores=NS,
    )

    @functools.partial(
        pl.kernel,
        out_type=jax.ShapeDtypeStruct((V // 2, 2 * D), jnp.float32),
        mesh=mesh,
        scratch_types=[
            [pltpu.VMEM((D, 128), jnp.float32)] * 2,
            [pltpu.VMEM((64, 2 * D), jnp.float32)] * 2,
            pltpu.VMEM((D, rem), jnp.float32),
            [pltpu.SemaphoreType.DMA] * 2,
            [pltpu.SemaphoreType.DMA] * 2,
        ],
        compiler_params=pltpu.CompilerParams(
            use_tc_tiling_on_sc=True, needs_layout_passes=False
        ),
    )
    def k(wt_hbm, wtail_hbm, out_hbm, inbs, outbs, intail, isems, osems):
        cid = lax.axis_index("c")
        sid = lax.axis_index("s")
        wid = sid * NC + cid

        ii = lax.iota(jnp.int32, L)
        perms = [lax.bitwise_and(ii + k_, L - 1) for k_ in range(L)]
        iihalf = lax.shift_right_logical(ii, 1)
        iipar = lax.shift_left(lax.bitwise_and(ii, 1), 6)

        def chunk_of(j):
            return wid + NW * j

        def issue_in(c, p):
            pltpu.async_copy(
                wt_hbm.at[:, pl.ds(pl.multiple_of(c * 128, 128), 128)],
                inbs[p],
                isems[p],
            )

        def wait_in(p):
            pltpu.make_async_copy(
                wt_hbm.at[:, pl.ds(0, 128)], inbs[p], isems[p]
            ).wait()

        def transpose(src, dst, w):
            @pl.loop(0, w, step=L)
            def _(rl0):
                pv = iihalf + lax.shift_right_logical(rl0, 1)
                for d0 in range(0, D, L):
                    for k_ in range(L):
                        dv = perms[k_] + d0
                        x = plsc.load_gather(src, [dv, rl0 + ii])
                        plsc.store_scatter(dst, [pv, iipar + dv], x)

        def issue_out(c, p):
            pltpu.async_copy(
                outbs[p],
                out_hbm.at[pl.ds(pl.multiple_of(c * 64, 64), 64)],
                osems[p],
            )

        def wait_out(p):
            pltpu.make_async_copy(
                outbs[p], out_hbm.at[pl.ds(0, 64)], osems[p]
            ).wait()

        def do_issue_in(j, p):
            c = chunk_of(j)

            @pl.when(c < n_full)
            def _():
                issue_in(c, p)

        do_issue_in(0, 0)
        do_issue_in(1, 1)

        @pl.loop(0, n_iter + (n_iter % 2), step=2)
        def _(o):
            for b in range(2):
                j = o + b
                p = b
                c = chunk_of(j)

                @pl.when(c < n_full)
                def _():
                    wait_in(p)

                    @pl.when(j >= 2)
                    def _():
                        wait_out(p)

                    transpose(inbs[p], outbs[p], 128)
                    issue_out(c, p)

                do_issue_in(j + 2, p)

        for b in range(2):
            wait_out(b)

        @pl.when(wid == NW - 1)
        def _():
            pltpu.sync_copy(wtail_hbm, intail)
            transpose(intail, outbs[0], rem)
            pltpu.sync_copy(
                outbs[0].at[pl.ds(0, rem // 2)],
                out_hbm.at[pl.ds(n_full * 64, rem // 2)],
            )

    return k(WT, Wtail)


@functools.partial(jax.jit, static_argnames=("B", "F", "D"))
def _emb_lookup(idxT_flat, W2, B, F, D):
    n_blk = B // BZ
    n_units = F * n_blk
    u_per_w = n_units // NW
    assert n_units % NW == 0 and u_per_w % 2 == 0

    mesh = plsc.VectorSubcoreMesh(
        core_axis_name="c", subcore_axis_name="s",
        num_cores=NC, num_subcores=NS,
    )

    @functools.partial(
        pl.kernel,
        out_type=jax.ShapeDtypeStruct((F, D, B), jnp.float32),
        mesh=mesh,
        scratch_types=[
            [pltpu.VMEM((BZ,), jnp.int32)] * 2,
            [pltpu.VMEM((BZ,), jnp.int32)] * 2,
            [pltpu.VMEM((BZ,), jnp.int32)] * 2,
            [pltpu.VMEM((BZ, 2 * D), jnp.float32)] * 2,
            [pltpu.VMEM((D, BZ), jnp.float32)] * 2,
            [pltpu.SemaphoreType.DMA] * 2,
            [pltpu.SemaphoreType.DMA] * 2,
            [pltpu.SemaphoreType.DMA] * 2,
        ],
        compiler_params=pltpu.CompilerParams(
            use_tc_tiling_on_sc=True, needs_layout_passes=False
        ),
    )
    def k(idx_hbm, table_hbm, out_hbm, idxr, idxh, parv, rows, bufTs,
          isems, gsems, osems):
        cid = lax.axis_index("c")
        sid = lax.axis_index("s")
        wid = sid * NC + cid
        u0 = wid * u_per_w

        def unit_fb(i):
            u = u0 + i
            f = lax.shift_right_logical(u, 6)
            b0 = pl.multiple_of(
                lax.shift_left(lax.bitwise_and(u, n_blk - 1), 8), BZ
            )
            return f, b0

        def issue_idx(i, p):
            f, b0 = unit_fb(i)
            pltpu.async_copy(
                idx_hbm.at[pl.ds(f * B + b0, BZ)], idxr[p], isems[p]
            )

        def wait_idx(p):
            pltpu.make_async_copy(
                idx_hbm.at[pl.ds(0, BZ)], idxr[p], isems[p]
            ).wait()

        def prep_idx(p):
            for t in range(BZ // L):
                v = idxr[p][pl.ds(t * L, L)]
                idxh[p][pl.ds(t * L, L)] = lax.shift_right_logical(v, 1)
                parv[p][pl.ds(t * L, L)] = lax.shift_left(
                    lax.bitwise_and(v, 1), 6
                )

        def issue_gather(p):
            for g in range(BZ // 128):
                pltpu.async_copy(
                    table_hbm.at[idxh[p].at[pl.ds(g * 128, 128)]],
                    rows[p].at[pl.ds(g * 128, 128)],
                    gsems[p],
                )

        def wait_gather(p):
            pltpu.make_async_copy(
                table_hbm.at[pl.ds(0, BZ)], rows[p], gsems[p]
            ).wait()

        def issue_out(i, p):
            f, b0 = unit_fb(i)
            pltpu.async_copy(
                bufTs[p], out_hbm.at[f, :, pl.ds(b0, BZ)], osems[p]
            )

        def wait_out(p):
            pltpu.make_async_copy(
                bufTs[p], out_hbm.at[0, :, pl.ds(0, BZ)], osems[p]
            ).wait()

        ii = lax.iota(jnp.int32, L)
        perms = [lax.bitwise_and(ii + k, L - 1) for k in range(L)]

        def transpose(p):
            @pl.loop(0, BZ, step=L)
            def _(r0):
                ridx = ii + r0
                par16 = parv[p][pl.ds(r0, L)]
                for d0 in range(0, D, L):
                    for k in range(L):
                        dst_d = perms[k] + d0
                        x = plsc.load_gather(rows[p], [ridx, dst_d + par16])
                        plsc.store_scatter(bufTs[p], [dst_d, ridx], x)

        issue_idx(0, 0)
        wait_idx(0)
        prep_idx(0)
        issue_gather(0)
        issue_idx(1, 1)

        @pl.loop(0, u_per_w, step=2)
        def _(o):
            for b in range(2):
                i = o + b
                p = b
                q = 1 - b

                @pl.when(i + 1 < u_per_w)
                def _():
                    wait_idx(q)
                    prep_idx(q)
                    issue_gather(q)

                wait_gather(p)

                @pl.when(i + 2 < u_per_w)
                def _():
                    issue_idx(i + 2, p)

                @pl.when(i >= 2)
                def _():
                    wait_out(p)

                transpose(p)
                issue_out(i, p)

        for p in range(2):
            wait_out(p)

    return k(idxT_flat, W2)


def kernel(input, W):
    B, F = input.shape
    D = W.shape[1]
    idxT_flat = input.T.reshape(-1)
    Wtail = W[(W.shape[0] // 128) * 128:, :].T
    W2 = _w_convert(W.T, Wtail)
    o = _emb_lookup(idxT_flat, W2, B, F, D)
    return jnp.transpose(o, (2, 0, 1))


# speedup vs baseline: 2.0346x; 2.0346x over previous
import functools

import jax
import jax.numpy as jnp
from jax import lax
from jax.experimental import pallas as pl
from jax.experimental.pallas import tpu as pltpu
from jax.experimental.pallas import tpu_sc as plsc

NC = 2
NS = 16
NW = NC * NS
BZ = 256
L = 16


def _w_convert(WT, Wtail):
    D, V = WT.shape
    n_full = V // 128
    rem = V - n_full * 128
    n_iter = (n_full + NW - 1) // NW

    mesh = plsc.VectorSubcoreMesh(
        core_axis_name="c", subcore_axis_name="s",
        num_cores=NC, num_subcores=NS,
    )

    @functools.partial(
        pl.kernel,
        out_type=jax.ShapeDtypeStruct((V // 2, 2 * D), jnp.float32),
        mesh=mesh,
        scratch_types=[
            [pltpu.VMEM((D, 128), jnp.float32)] * 2,
            [pltpu.VMEM((64, 2 * D), jnp.float32)] * 2,
            pltpu.VMEM((D, rem), jnp.float32),
            [pltpu.SemaphoreType.DMA] * 2,
            [pltpu.SemaphoreType.DMA] * 2,
        ],
        compiler_params=pltpu.CompilerParams(
            use_tc_tiling_on_sc=True, needs_layout_passes=False
        ),
    )
    def k(wt_hbm, wtail_hbm, out_hbm, inbs, outbs, intail, isems, osems):
        cid = lax.axis_index("c")
        sid = lax.axis_index("s")
        wid = sid * NC + cid

        ii = lax.iota(jnp.int32, L)
        perms = [lax.bitwise_and(ii + k_, L - 1) for k_ in range(L)]
        iihalf = lax.shift_right_logical(ii, 1)
        iipar = lax.shift_left(lax.bitwise_and(ii, 1), 6)

        def chunk_of(j):
            return wid + NW * j

        def issue_in(c, p):
            pltpu.async_copy(
                wt_hbm.at[:, pl.ds(pl.multiple_of(c * 128, 128), 128)],
                inbs[p],
                isems[p],
            )

        def wait_in(p):
            pltpu.make_async_copy(
                wt_hbm.at[:, pl.ds(0, 128)], inbs[p], isems[p]
            ).wait()

        def transpose(src, dst, w):
            @pl.loop(0, w, step=L)
            def _(rl0):
                pv = iihalf + lax.shift_right_logical(rl0, 1)
                rv = rl0 + ii
                for d0 in range(0, D, L):
                    xs = [
                        plsc.load_gather(src, [perms[k_] + d0, rv])
                        for k_ in range(L)
                    ]
                    for k_ in range(L):
                        plsc.store_scatter(
                            dst, [pv, iipar + perms[k_] + d0], xs[k_]
                        )

        def issue_out(c, p):
            pltpu.async_copy(
                outbs[p],
                out_hbm.at[pl.ds(pl.multiple_of(c * 64, 64), 64)],
                osems[p],
            )

        def wait_out(p):
            pltpu.make_async_copy(
                outbs[p], out_hbm.at[pl.ds(0, 64)], osems[p]
            ).wait()

        def do_issue_in(j, p):
            c = chunk_of(j)

            @pl.when(c < n_full)
            def _():
                issue_in(c, p)

        do_issue_in(0, 0)
        do_issue_in(1, 1)

        @pl.loop(0, n_iter + (n_iter % 2), step=2)
        def _(o):
            for b in range(2):
                j = o + b
                p = b
                c = chunk_of(j)

                @pl.when(c < n_full)
                def _():
                    wait_in(p)

                    @pl.when(j >= 2)
                    def _():
                        wait_out(p)

                    transpose(inbs[p], outbs[p], 128)
                    issue_out(c, p)

                do_issue_in(j + 2, p)

        for b in range(2):
            wait_out(b)

        @pl.when(wid == NW - 1)
        def _():
            pltpu.sync_copy(wtail_hbm, intail)
            transpose(intail, outbs[0], rem)
            pltpu.sync_copy(
                outbs[0].at[pl.ds(0, rem // 2)],
                out_hbm.at[pl.ds(n_full * 64, rem // 2)],
            )

    return k(WT, Wtail)


@functools.partial(jax.jit, static_argnames=("B", "F", "D"))
def _emb_lookup(idxT_flat, W2, B, F, D):
    n_blk = B // BZ
    n_units = F * n_blk
    u_per_w = n_units // NW
    assert n_units % NW == 0 and u_per_w % 2 == 0

    mesh = plsc.VectorSubcoreMesh(
        core_axis_name="c", subcore_axis_name="s",
        num_cores=NC, num_subcores=NS,
    )

    @functools.partial(
        pl.kernel,
        out_type=jax.ShapeDtypeStruct((F, D, B), jnp.float32),
        mesh=mesh,
        scratch_types=[
            [pltpu.VMEM((BZ,), jnp.int32)] * 2,
            [pltpu.VMEM((BZ,), jnp.int32)] * 2,
            [pltpu.VMEM((BZ,), jnp.int32)] * 2,
            [pltpu.VMEM((BZ, 2 * D), jnp.float32)] * 2,
            [pltpu.VMEM((D, BZ), jnp.float32)] * 2,
            [pltpu.SemaphoreType.DMA] * 2,
            [pltpu.SemaphoreType.DMA] * 2,
            [pltpu.SemaphoreType.DMA] * 2,
        ],
        compiler_params=pltpu.CompilerParams(
            use_tc_tiling_on_sc=True, needs_layout_passes=False
        ),
    )
    def k(idx_hbm, table_hbm, out_hbm, idxr, idxh, parv, rows, bufTs,
          isems, gsems, osems):
        cid = lax.axis_index("c")
        sid = lax.axis_index("s")
        wid = sid * NC + cid
        u0 = wid * u_per_w

        def unit_fb(i):
            u = u0 + i
            f = lax.shift_right_logical(u, 6)
            b0 = pl.multiple_of(
                lax.shift_left(lax.bitwise_and(u, n_blk - 1), 8), BZ
            )
            return f, b0

        def issue_idx(i, p):
            f, b0 = unit_fb(i)
            pltpu.async_copy(
                idx_hbm.at[pl.ds(f * B + b0, BZ)], idxr[p], isems[p]
            )

        def wait_idx(p):
            pltpu.make_async_copy(
                idx_hbm.at[pl.ds(0, BZ)], idxr[p], isems[p]
            ).wait()

        def prep_idx(p):
            for t in range(BZ // L):
                v = idxr[p][pl.ds(t * L, L)]
                idxh[p][pl.ds(t * L, L)] = lax.shift_right_logical(v, 1)
                parv[p][pl.ds(t * L, L)] = lax.shift_left(
                    lax.bitwise_and(v, 1), 6
                )

        def issue_gather(p):
            for g in range(BZ // 128):
                pltpu.async_copy(
                    table_hbm.at[idxh[p].at[pl.ds(g * 128, 128)]],
                    rows[p].at[pl.ds(g * 128, 128)],
                    gsems[p],
                )

        def wait_gather(p):
            pltpu.make_async_copy(
                table_hbm.at[pl.ds(0, BZ)], rows[p], gsems[p]
            ).wait()

        def issue_out(i, p):
            f, b0 = unit_fb(i)
            pltpu.async_copy(
                bufTs[p], out_hbm.at[f, :, pl.ds(b0, BZ)], osems[p]
            )

        def wait_out(p):
            pltpu.make_async_copy(
                bufTs[p], out_hbm.at[0, :, pl.ds(0, BZ)], osems[p]
            ).wait()

        ii = lax.iota(jnp.int32, L)
        perms = [lax.bitwise_and(ii + k, L - 1) for k in range(L)]

        def transpose(p):
            @pl.loop(0, BZ, step=L)
            def _(r0):
                ridx = ii + r0
                par16 = parv[p][pl.ds(r0, L)]
                for d0 in range(0, D, L):
                    xs = [
                        plsc.load_gather(
                            rows[p], [ridx, perms[k] + d0 + par16]
                        )
                        for k in range(L)
                    ]
                    for k in range(L):
                        plsc.store_scatter(
                            bufTs[p], [perms[k] + d0, ridx], xs[k]
                        )

        issue_idx(0, 0)
        wait_idx(0)
        prep_idx(0)
        issue_gather(0)
        issue_idx(1, 1)

        @pl.loop(0, u_per_w, step=2)
        def _(o):
            for b in range(2):
                i = o + b
                p = b
                q = 1 - b

                @pl.when(i + 1 < u_per_w)
                def _():
                    wait_idx(q)
                    prep_idx(q)
                    issue_gather(q)

                wait_gather(p)

                @pl.when(i + 2 < u_per_w)
                def _():
                    issue_idx(i + 2, p)

                @pl.when(i >= 2)
                def _():
                    wait_out(p)

                transpose(p)
                issue_out(i, p)

        for p in range(2):
            wait_out(p)

    return k(idxT_flat, W2)


def kernel(input, W):
    B, F = input.shape
    D = W.shape[1]
    idxT_flat = input.T.reshape(-1)
    Wtail = W[(W.shape[0] // 128) * 128:, :].T
    W2 = _w_convert(W.T, Wtail)
    o = _emb_lookup(idxT_flat, W2, B, F, D)
    return jnp.transpose(o, (2, 0, 1))


# speedup vs baseline: 2.5246x; 1.2408x over previous
import functools

import jax
import jax.numpy as jnp
from jax import lax
from jax.experimental import pallas as pl
from jax.experimental.pallas import tpu as pltpu
from jax.experimental.pallas import tpu_sc as plsc

NC = 2
NS = 16
NW = NC * NS
BZ = 256
L = 16


def _w_convert(WT, Wtail):
    D, V = WT.shape
    CW = 256
    n_full = V // CW
    rem = V - n_full * CW
    n_iter = (n_full + NW - 1) // NW

    mesh = plsc.VectorSubcoreMesh(
        core_axis_name="c", subcore_axis_name="s",
        num_cores=NC, num_subcores=NS,
    )

    @functools.partial(
        pl.kernel,
        out_type=jax.ShapeDtypeStruct((V // 2, 2 * D), jnp.float32),
        mesh=mesh,
        scratch_types=[
            [pltpu.VMEM((D, CW), jnp.float32)] * 2,
            [pltpu.VMEM((CW // 2, 2 * D), jnp.float32)] * 2,
            pltpu.VMEM((D, rem), jnp.float32),
            [pltpu.SemaphoreType.DMA] * 2,
            [pltpu.SemaphoreType.DMA] * 2,
        ],
        compiler_params=pltpu.CompilerParams(
            use_tc_tiling_on_sc=True, needs_layout_passes=False
        ),
    )
    def k(wt_hbm, wtail_hbm, out_hbm, inbs, outbs, intail, isems, osems):
        cid = lax.axis_index("c")
        sid = lax.axis_index("s")
        wid = sid * NC + cid

        ii = lax.iota(jnp.int32, L)
        perms = [lax.bitwise_and(ii + k_, L - 1) for k_ in range(L)]
        iihalf = lax.shift_right_logical(ii, 1)
        iipar = lax.shift_left(lax.bitwise_and(ii, 1), 6)

        def chunk_of(j):
            return wid + NW * j

        def issue_in(c, p):
            pltpu.async_copy(
                wt_hbm.at[:, pl.ds(pl.multiple_of(c * CW, CW), CW)],
                inbs[p],
                isems[p],
            )

        def wait_in(p):
            pltpu.make_async_copy(
                wt_hbm.at[:, pl.ds(0, CW)], inbs[p], isems[p]
            ).wait()

        def transpose(src, dst, w):
            @pl.loop(0, w, step=L)
            def _(rl0):
                pv = iihalf + lax.shift_right_logical(rl0, 1)
                rv = rl0 + ii
                for d0 in range(0, D, L):
                    xs = [
                        plsc.load_gather(src, [perms[k_] + d0, rv])
                        for k_ in range(L)
                    ]
                    for k_ in range(L):
                        plsc.store_scatter(
                            dst, [pv, iipar + perms[k_] + d0], xs[k_]
                        )

        def issue_out(c, p):
            pltpu.async_copy(
                outbs[p],
                out_hbm.at[pl.ds(pl.multiple_of(c * (CW // 2), CW // 2),
                                 CW // 2)],
                osems[p],
            )

        def wait_out(p):
            pltpu.make_async_copy(
                outbs[p], out_hbm.at[pl.ds(0, CW // 2)], osems[p]
            ).wait()

        def do_issue_in(j, p):
            c = chunk_of(j)

            @pl.when(c < n_full)
            def _():
                issue_in(c, p)

        do_issue_in(0, 0)
        do_issue_in(1, 1)

        @pl.loop(0, n_iter + (n_iter % 2), step=2)
        def _(o):
            for b in range(2):
                j = o + b
                p = b
                c = chunk_of(j)

                @pl.when(c < n_full)
                def _():
                    wait_in(p)

                    @pl.when(j >= 2)
                    def _():
                        wait_out(p)

                    transpose(inbs[p], outbs[p], CW)
                    issue_out(c, p)

                do_issue_in(j + 2, p)

        for b in range(2):
            wait_out(b)

        @pl.when(wid == NW - 1)
        def _():
            pltpu.sync_copy(wtail_hbm, intail)
            transpose(intail, outbs[0], rem)
            pltpu.sync_copy(
                outbs[0].at[pl.ds(0, rem // 2)],
                out_hbm.at[pl.ds(n_full * (CW // 2), rem // 2)],
            )

    return k(WT, Wtail)


@functools.partial(jax.jit, static_argnames=("B", "F", "D"))
def _emb_lookup(idxT_flat, W2, B, F, D):
    n_blk = B // BZ
    n_units = F * n_blk
    u_per_w = n_units // NW
    assert n_units % NW == 0 and u_per_w % 2 == 0

    mesh = plsc.VectorSubcoreMesh(
        core_axis_name="c", subcore_axis_name="s",
        num_cores=NC, num_subcores=NS,
    )

    @functools.partial(
        pl.kernel,
        out_type=jax.ShapeDtypeStruct((F, D, B), jnp.float32),
        mesh=mesh,
        scratch_types=[
            [pltpu.VMEM((BZ,), jnp.int32)] * 2,
            [pltpu.VMEM((BZ,), jnp.int32)] * 2,
            [pltpu.VMEM((BZ,), jnp.int32)] * 2,
            [pltpu.VMEM((BZ, 2 * D), jnp.float32)] * 2,
            [pltpu.VMEM((D, BZ), jnp.float32)] * 2,
            [pltpu.SemaphoreType.DMA] * 2,
            [pltpu.SemaphoreType.DMA] * 2,
            [pltpu.SemaphoreType.DMA] * 2,
        ],
        compiler_params=pltpu.CompilerParams(
            use_tc_tiling_on_sc=True, needs_layout_passes=False
        ),
    )
    def k(idx_hbm, table_hbm, out_hbm, idxr, idxh, parv, rows, bufTs,
          isems, gsems, osems):
        cid = lax.axis_index("c")
        sid = lax.axis_index("s")
        wid = sid * NC + cid
        u0 = wid * u_per_w

        def unit_fb(i):
            u = u0 + i
            f = lax.shift_right_logical(u, 6)
            b0 = pl.multiple_of(
                lax.shift_left(lax.bitwise_and(u, n_blk - 1), 8), BZ
            )
            return f, b0

        def issue_idx(i, p):
            f, b0 = unit_fb(i)
            pltpu.async_copy(
                idx_hbm.at[pl.ds(f * B + b0, BZ)], idxr[p], isems[p]
            )

        def wait_idx(p):
            pltpu.make_async_copy(
                idx_hbm.at[pl.ds(0, BZ)], idxr[p], isems[p]
            ).wait()

        def prep_idx(p):
            for t in range(BZ // L):
                v = idxr[p][pl.ds(t * L, L)]
                idxh[p][pl.ds(t * L, L)] = lax.shift_right_logical(v, 1)
                parv[p][pl.ds(t * L, L)] = lax.shift_left(
                    lax.bitwise_and(v, 1), 6
                )

        def issue_gather(p):
            for g in range(BZ // 128):
                pltpu.async_copy(
                    table_hbm.at[idxh[p].at[pl.ds(g * 128, 128)]],
                    rows[p].at[pl.ds(g * 128, 128)],
                    gsems[p],
                )

        def wait_gather(p):
            pltpu.make_async_copy(
                table_hbm.at[pl.ds(0, BZ)], rows[p], gsems[p]
            ).wait()

        def issue_out(i, p):
            f, b0 = unit_fb(i)
            pltpu.async_copy(
                bufTs[p], out_hbm.at[f, :, pl.ds(b0, BZ)], osems[p]
            )

        def wait_out(p):
            pltpu.make_async_copy(
                bufTs[p], out_hbm.at[0, :, pl.ds(0, BZ)], osems[p]
            ).wait()

        ii = lax.iota(jnp.int32, L)
        perms = [lax.bitwise_and(ii + k, L - 1) for k in range(L)]

        def transpose(p):
            @pl.loop(0, BZ, step=L)
            def _(r0):
                ridx = ii + r0
                par16 = parv[p][pl.ds(r0, L)]
                for d0 in range(0, D, L):
                    xs = [
                        plsc.load_gather(
                            rows[p], [ridx, perms[k] + d0 + par16]
                        )
                        for k in range(L)
                    ]
                    for k in range(L):
                        plsc.store_scatter(
                            bufTs[p], [perms[k] + d0, ridx], xs[k]
                        )

        issue_idx(0, 0)
        wait_idx(0)
        prep_idx(0)
        issue_gather(0)
        issue_idx(1, 1)

        @pl.loop(0, u_per_w, step=2)
        def _(o):
            for b in range(2):
                i = o + b
                p = b
                q = 1 - b

                @pl.when(i + 1 < u_per_w)
                def _():
                    wait_idx(q)
                    prep_idx(q)
                    issue_gather(q)

                wait_gather(p)

                @pl.when(i + 2 < u_per_w)
                def _():
                    issue_idx(i + 2, p)

                @pl.when(i >= 2)
                def _():
                    wait_out(p)

                transpose(p)
                issue_out(i, p)

        for p in range(2):
            wait_out(p)

    return k(idxT_flat, W2)


def kernel(input, W):
    B, F = input.shape
    D = W.shape[1]
    idxT_flat = input.T.reshape(-1)
    Wtail = W[(W.shape[0] // 128) * 128:, :].T
    W2 = _w_convert(W.T, Wtail)
    o = _emb_lookup(idxT_flat, W2, B, F, D)
    return jnp.transpose(o, (2, 0, 1))


# speedup vs baseline: 2.5542x; 1.0118x over previous
import functools

import jax
import jax.numpy as jnp
from jax import lax
from jax.experimental import pallas as pl
from jax.experimental.pallas import tpu as pltpu
from jax.experimental.pallas import tpu_sc as plsc

NC = 2
NS = 16
NW = NC * NS
BZ = 256
L = 16


def _w_convert(WT, Wtail):
    D, V = WT.shape
    CW = 384
    n_full = V // CW
    rem = V - n_full * CW
    n_iter = (n_full + NW - 1) // NW

    mesh = plsc.VectorSubcoreMesh(
        core_axis_name="c", subcore_axis_name="s",
        num_cores=NC, num_subcores=NS,
    )

    @functools.partial(
        pl.kernel,
        out_type=jax.ShapeDtypeStruct((V // 2, 2 * D), jnp.float32),
        mesh=mesh,
        scratch_types=[
            [pltpu.VMEM((D, CW), jnp.float32)] * 2,
            [pltpu.VMEM((CW // 2, 2 * D), jnp.float32)] * 2,
            pltpu.VMEM((D, rem), jnp.float32),
            [pltpu.SemaphoreType.DMA] * 2,
            [pltpu.SemaphoreType.DMA] * 2,
        ],
        compiler_params=pltpu.CompilerParams(
            use_tc_tiling_on_sc=True, needs_layout_passes=False
        ),
    )
    def k(wt_hbm, wtail_hbm, out_hbm, inbs, outbs, intail, isems, osems):
        cid = lax.axis_index("c")
        sid = lax.axis_index("s")
        wid = sid * NC + cid

        ii = lax.iota(jnp.int32, L)
        perms = [lax.bitwise_and(ii + k_, L - 1) for k_ in range(L)]
        iihalf = lax.shift_right_logical(ii, 1)
        iipar = lax.shift_left(lax.bitwise_and(ii, 1), 6)

        def chunk_of(j):
            return wid + NW * j

        def issue_in(c, p):
            pltpu.async_copy(
                wt_hbm.at[:, pl.ds(pl.multiple_of(c * CW, CW), CW)],
                inbs[p],
                isems[p],
            )

        def wait_in(p):
            pltpu.make_async_copy(
                wt_hbm.at[:, pl.ds(0, CW)], inbs[p], isems[p]
            ).wait()

        def transpose(src, dst, w):
            @pl.loop(0, w, step=L)
            def _(rl0):
                pv = iihalf + lax.shift_right_logical(rl0, 1)
                rv = rl0 + ii
                for d0 in range(0, D, L):
                    xs = [
                        plsc.load_gather(src, [perms[k_] + d0, rv])
                        for k_ in range(L)
                    ]
                    for k_ in range(L):
                        plsc.store_scatter(
                            dst, [pv, iipar + perms[k_] + d0], xs[k_]
                        )

        def issue_out(c, p):
            pltpu.async_copy(
                outbs[p],
                out_hbm.at[pl.ds(pl.multiple_of(c * (CW // 2), CW // 2),
                                 CW // 2)],
                osems[p],
            )

        def wait_out(p):
            pltpu.make_async_copy(
                outbs[p], out_hbm.at[pl.ds(0, CW // 2)], osems[p]
            ).wait()

        def do_issue_in(j, p):
            c = chunk_of(j)

            @pl.when(c < n_full)
            def _():
                issue_in(c, p)

        do_issue_in(0, 0)
        do_issue_in(1, 1)

        @pl.loop(0, n_iter + (n_iter % 2), step=2)
        def _(o):
            for b in range(2):
                j = o + b
                p = b
                c = chunk_of(j)

                @pl.when(c < n_full)
                def _():
                    wait_in(p)

                    @pl.when(j >= 2)
                    def _():
                        wait_out(p)

                    transpose(inbs[p], outbs[p], CW)
                    issue_out(c, p)

                do_issue_in(j + 2, p)

        for b in range(2):
            wait_out(b)

        @pl.when(wid == NW - 1)
        def _():
            pltpu.sync_copy(wtail_hbm, intail)
            transpose(intail, outbs[0], rem)
            pltpu.sync_copy(
                outbs[0].at[pl.ds(0, rem // 2)],
                out_hbm.at[pl.ds(n_full * (CW // 2), rem // 2)],
            )

    return k(WT, Wtail)


@functools.partial(jax.jit, static_argnames=("B", "F", "D"))
def _emb_lookup(idxT_flat, W2, B, F, D):
    n_blk = B // BZ
    n_units = F * n_blk
    u_per_w = n_units // NW
    assert n_units % NW == 0 and u_per_w % 2 == 0

    mesh = plsc.VectorSubcoreMesh(
        core_axis_name="c", subcore_axis_name="s",
        num_cores=NC, num_subcores=NS,
    )

    @functools.partial(
        pl.kernel,
        out_type=jax.ShapeDtypeStruct((F, D, B), jnp.float32),
        mesh=mesh,
        scratch_types=[
            [pltpu.VMEM((BZ,), jnp.int32)] * 2,
            [pltpu.VMEM((BZ,), jnp.int32)] * 2,
            [pltpu.VMEM((BZ,), jnp.int32)] * 2,
            [pltpu.VMEM((BZ, 2 * D), jnp.float32)] * 2,
            [pltpu.VMEM((D, BZ), jnp.float32)] * 2,
            [pltpu.SemaphoreType.DMA] * 2,
            [pltpu.SemaphoreType.DMA] * 2,
            [pltpu.SemaphoreType.DMA] * 2,
        ],
        compiler_params=pltpu.CompilerParams(
            use_tc_tiling_on_sc=True, needs_layout_passes=False
        ),
    )
    def k(idx_hbm, table_hbm, out_hbm, idxr, idxh, parv, rows, bufTs,
          isems, gsems, osems):
        cid = lax.axis_index("c")
        sid = lax.axis_index("s")
        wid = sid * NC + cid
        u0 = wid * u_per_w

        def unit_fb(i):
            u = u0 + i
            f = lax.shift_right_logical(u, 6)
            b0 = pl.multiple_of(
                lax.shift_left(lax.bitwise_and(u, n_blk - 1), 8), BZ
            )
            return f, b0

        def issue_idx(i, p):
            f, b0 = unit_fb(i)
            pltpu.async_copy(
                idx_hbm.at[pl.ds(f * B + b0, BZ)], idxr[p], isems[p]
            )

        def wait_idx(p):
            pltpu.make_async_copy(
                idx_hbm.at[pl.ds(0, BZ)], idxr[p], isems[p]
            ).wait()

        def prep_idx(p):
            for t in range(BZ // L):
                v = idxr[p][pl.ds(t * L, L)]
                idxh[p][pl.ds(t * L, L)] = lax.shift_right_logical(v, 1)
                parv[p][pl.ds(t * L, L)] = lax.shift_left(
                    lax.bitwise_and(v, 1), 6
                )

        def issue_gather(p):
            for g in range(BZ // 128):
                pltpu.async_copy(
                    table_hbm.at[idxh[p].at[pl.ds(g * 128, 128)]],
                    rows[p].at[pl.ds(g * 128, 128)],
                    gsems[p],
                )

        def wait_gather(p):
            pltpu.make_async_copy(
                table_hbm.at[pl.ds(0, BZ)], rows[p], gsems[p]
            ).wait()

        def issue_out(i, p):
            f, b0 = unit_fb(i)
            pltpu.async_copy(
                bufTs[p], out_hbm.at[f, :, pl.ds(b0, BZ)], osems[p]
            )

        def wait_out(p):
            pltpu.make_async_copy(
                bufTs[p], out_hbm.at[0, :, pl.ds(0, BZ)], osems[p]
            ).wait()

        ii = lax.iota(jnp.int32, L)
        perms = [lax.bitwise_and(ii + k, L - 1) for k in range(L)]

        def transpose(p):
            @pl.loop(0, BZ, step=L)
            def _(r0):
                ridx = ii + r0
                par16 = parv[p][pl.ds(r0, L)]
                for d0 in range(0, D, L):
                    xs = [
                        plsc.load_gather(
                            rows[p], [ridx, perms[k] + d0 + par16]
                        )
                        for k in range(L)
                    ]
                    for k in range(L):
                        plsc.store_scatter(
                            bufTs[p], [perms[k] + d0, ridx], xs[k]
                        )

        issue_idx(0, 0)
        wait_idx(0)
        prep_idx(0)
        issue_gather(0)
        issue_idx(1, 1)

        @pl.loop(0, u_per_w, step=2)
        def _(o):
            for b in range(2):
                i = o + b
                p = b
                q = 1 - b

                @pl.when(i + 1 < u_per_w)
                def _():
                    wait_idx(q)
                    prep_idx(q)
                    issue_gather(q)

                wait_gather(p)

                @pl.when(i + 2 < u_per_w)
                def _():
                    issue_idx(i + 2, p)

                @pl.when(i >= 2)
                def _():
                    wait_out(p)

                transpose(p)
                issue_out(i, p)

        for p in range(2):
            wait_out(p)

    return k(idxT_flat, W2)


def kernel(input, W):
    B, F = input.shape
    D = W.shape[1]
    idxT_flat = input.T.reshape(-1)
    Wtail = W[(W.shape[0] // 128) * 128:, :].T
    W2 = _w_convert(W.T, Wtail)
    o = _emb_lookup(idxT_flat, W2, B, F, D)
    return jnp.transpose(o, (2, 0, 1))
